# Initial kernel scaffold; baseline (speedup 1.0000x reference)
import jax
import jax.numpy as jnp
from jax.experimental import pallas as pl


def kernel(x, mask1, mask2, w1, b1, w2, b2, ws, bs, gamma, beta):
    raise NotImplementedError("write your pallas kernel here")


# speedup vs baseline: 2.0517x; 2.0517x over previous
import functools

import jax
import jax.numpy as jnp
from jax.experimental import pallas as pl
from jax.experimental.pallas import tpu as pltpu


def _conv1_band_kernel(xe_ref, xee_ref, xo_ref, xoe_ref, w_ref, b_ref,
                       o_ref, *, rows, neg_slope):
    wo = o_ref.shape[3]
    xe = jnp.concatenate([xe_ref[0], xee_ref[0]], axis=0)
    xo = jnp.concatenate([xo_ref[0], xoe_ref[0]], axis=0)
    cols = []
    for r in range(rows):
        parts = []
        for kh in range(3):
            e = xe[2 * r + kh]
            o = xo[2 * r + kh]
            parts.append(e[:, 0:wo])
            parts.append(o[:, 0:wo])
            parts.append(e[:, 1:wo + 1])
        cols.append(jnp.concatenate(parts, axis=0))
    rhs = jnp.concatenate(cols, axis=1)
    acc = jnp.dot(w_ref[...], rhs, preferred_element_type=jnp.float32)
    acc = acc + b_ref[...]
    acc = jnp.where(acc >= 0.0, acc, acc * neg_slope)
    accb = acc.astype(o_ref.dtype)
    for r in range(rows):
        o_ref[0, r] = accb[:, r * wo:(r + 1) * wo]


def _tail_band_kernel(y_ref, ytop_ref, ybot_ref, xo_ref,
                      w2_ref, ws_ref, g_ref, b2_ref, bs_ref, beta_ref,
                      o_ref, *, rows, nb):
    cout, wo = o_ref.shape[2], o_ref.shape[3]
    b = pl.program_id(1)
    tm = (b > 0).astype(y_ref.dtype)
    bm = (b < nb - 1).astype(y_ref.dtype)
    y = jnp.concatenate([ytop_ref[0] * tm, y_ref[0], ybot_ref[0] * bm],
                        axis=0)
    zc = jnp.zeros((rows + 2, cout, 1), y.dtype)
    yp = jnp.concatenate([zc, y, zc], axis=2)
    cols = []
    scols = []
    for r in range(rows):
        parts = []
        for kh in range(3):
            row = yp[r + kh]
            for kw in range(3):
                parts.append(row[:, kw:kw + wo])
        cols.append(jnp.concatenate(parts, axis=0))
        scols.append(xo_ref[0, 2 * r + 1][:, 0:wo])
    rhs = jnp.concatenate(cols, axis=1)
    srhs = jnp.concatenate(scols, axis=1)
    y2 = jnp.dot(w2_ref[...], rhs,
                 preferred_element_type=jnp.float32) + b2_ref[...]
    ident = jnp.dot(ws_ref[...], srhs,
                    preferred_element_type=jnp.float32) + bs_ref[...]
    ysq = (y2 * y2).astype(g_ref.dtype)
    norm = jnp.dot(g_ref[...], ysq,
                   preferred_element_type=jnp.float32) + beta_ref[...]
    out = y2 * jax.lax.rsqrt(norm) + ident
    for r in range(rows):
        o_ref[0, r] = out[:, r * wo:(r + 1) * wo]


def _pick_band(ho):
    for r in (16, 8, 4, 2, 1):
        if ho % r == 0:
            return r
    return 1


def kernel(x, mask1, mask2, w1, b1, w2, b2, ws, bs, gamma, beta):
    del mask1, mask2
    n, cin, h, w = x.shape
    cout = w1.shape[1]
    ho = (h - 1) // 2 + 1
    wo = (w - 1) // 2 + 1
    rows = _pick_band(ho)
    nb = ho // rows

    xb = jnp.pad(jnp.transpose(x.astype(jnp.bfloat16), (0, 2, 1, 3)),
                 ((0, 0), (1, 1), (0, 0), (1, 1)))
    xpe = xb[..., 0::2]
    xpo = xb[..., 1::2]
    we, wod = xpe.shape[-1], xpo.shape[-1]

    w1l = jnp.transpose(w1, (1, 0, 2)).reshape(cout, 9 * cin).astype(jnp.bfloat16)
    w2l = jnp.transpose(w2, (1, 0, 2)).reshape(cout, 9 * cout).astype(jnp.bfloat16)
    wsl = ws.astype(jnp.bfloat16)
    gl = gamma.astype(jnp.bfloat16)

    cparams = pltpu.CompilerParams(
        dimension_semantics=("parallel", "parallel"),
        vmem_limit_bytes=64 * 1024 * 1024,
    )

    def const_spec(shape):
        return pl.BlockSpec(shape, lambda i, b: (0,) * len(shape))

    out1 = pl.pallas_call(
        functools.partial(_conv1_band_kernel, rows=rows, neg_slope=0.01),
        out_shape=jax.ShapeDtypeStruct((n, ho, cout, wo), jnp.bfloat16),
        grid=(n, nb),
        in_specs=[
            pl.BlockSpec((1, 2 * rows, cin, we), lambda i, b: (i, b, 0, 0)),
            pl.BlockSpec((1, 1, cin, we),
                         lambda i, b: (i, 2 * rows * (b + 1), 0, 0)),
            pl.BlockSpec((1, 2 * rows, cin, wod), lambda i, b: (i, b, 0, 0)),
            pl.BlockSpec((1, 1, cin, wod),
                         lambda i, b: (i, 2 * rows * (b + 1), 0, 0)),
            const_spec((cout, 9 * cin)),
            const_spec((cout, 1)),
        ],
        out_specs=pl.BlockSpec((1, rows, cout, wo), lambda i, b: (i, b, 0, 0)),
        compiler_params=cparams,
    )(xpe, xpe, xpo, xpo, w1l, b1)

    out2 = pl.pallas_call(
        functools.partial(_tail_band_kernel, rows=rows, nb=nb),
        out_shape=jax.ShapeDtypeStruct((n, ho, cout, wo), jnp.float32),
        grid=(n, nb),
        in_specs=[
            pl.BlockSpec((1, rows, cout, wo), lambda i, b: (i, b, 0, 0)),
            pl.BlockSpec((1, 1, cout, wo),
                         lambda i, b: (i, jnp.maximum(rows * b - 1, 0), 0, 0)),
            pl.BlockSpec((1, 1, cout, wo),
                         lambda i, b: (i, jnp.minimum(rows * (b + 1), ho - 1),
                                       0, 0)),
            pl.BlockSpec((1, 2 * rows, cin, wod), lambda i, b: (i, b, 0, 0)),
            const_spec((cout, 9 * cout)),
            const_spec((cout, cin)),
            const_spec((cout, cout)),
            const_spec((cout, 1)),
            const_spec((cout, 1)),
            const_spec((cout, 1)),
        ],
        out_specs=pl.BlockSpec((1, rows, cout, wo), lambda i, b: (i, b, 0, 0)),
        compiler_params=cparams,
    )(out1, out1, out1, xpo, w2l, wsl, gl, b2, bs, beta)

    return jnp.transpose(out2, (0, 2, 1, 3))


# speedup vs baseline: 19.4899x; 9.4995x over previous
import functools

import jax
import jax.numpy as jnp
from jax.experimental import pallas as pl
from jax.experimental.pallas import tpu as pltpu


def _fused_band_kernel(x_ref, xt0_ref, xt1_ref, xt2_ref, xb0_ref, xb1_ref,
                       s_ref, w1_ref, w2_ref, ws_ref, g_ref,
                       b1_ref, b2_ref, bs_ref, beta_ref,
                       o_ref, *, rows, nb, w, wo, neg_slope):
    b = pl.program_id(1)
    bf = jnp.bfloat16
    cin = x_ref.shape[0]

    xm = x_ref[...].astype(bf)
    pieces = [xt0_ref[...].astype(bf), xt1_ref[...].astype(bf),
              xt2_ref[...].astype(bf)]
    pieces += [xm[:, l * w:(l + 1) * w] for l in range(2 * rows)]
    pieces += [xb0_ref[...].astype(bf), xb1_ref[...].astype(bf)]
    xs = jnp.concatenate(pieces, axis=0)
    sel = jnp.dot(xs, s_ref[...],
                  preferred_element_type=jnp.float32).astype(bf)

    zero_mask = (b > 0).astype(bf)

    def row_eo(l):
        base = (l + 3) * cin
        blk = sel[base:base + cin]
        return blk[:, 0:wo], blk[:, wo:2 * wo]

    zc = None

    def shift_r(v):
        return jnp.concatenate([zc, v[:, 0:wo - 1]], axis=1)

    zc = jnp.zeros((cin, 1), bf)
    cols = []
    for jj in range(-1, rows + 1):
        parts = []
        for kh in range(3):
            l = 2 * jj + kh - 1
            e, o = row_eo(l)
            if l == -1:
                e = e * zero_mask
                o = o * zero_mask
            parts.append(shift_r(o))
            parts.append(e)
            parts.append(o)
        cols.append(jnp.concatenate(parts, axis=0))
    rhs1 = jnp.concatenate(cols, axis=1)
    y1 = jnp.dot(w1_ref[...], rhs1,
                 preferred_element_type=jnp.float32) + b1_ref[...]
    y1 = jnp.where(y1 >= 0.0, y1, y1 * neg_slope)

    cout = y1.shape[0]
    tm = (b > 0).astype(jnp.float32)
    bm = (b < nb - 1).astype(jnp.float32)
    y1 = y1 * jnp.concatenate(
        [jnp.full((1, wo), tm), jnp.ones((1, rows * wo)),
         jnp.full((1, wo), bm)], axis=1)
    y1b = y1.astype(bf)

    zc2 = jnp.zeros((cout, 1), bf)
    cols2 = []
    for r in range(rows):
        parts = []
        for kh in range(3):
            g = y1b[:, (r + kh) * wo:(r + kh + 1) * wo]
            parts.append(jnp.concatenate([zc2, g[:, 0:wo - 1]], axis=1))
            parts.append(g)
            parts.append(jnp.concatenate([g[:, 1:wo], zc2], axis=1))
        cols2.append(jnp.concatenate(parts, axis=0))
    rhs2 = jnp.concatenate(cols2, axis=1)
    y2 = jnp.dot(w2_ref[...], rhs2,
                 preferred_element_type=jnp.float32) + b2_ref[...]

    srhs = jnp.concatenate(
        [row_eo(2 * r)[0] for r in range(rows)], axis=1)
    ident = jnp.dot(ws_ref[...], srhs,
                    preferred_element_type=jnp.float32) + bs_ref[...]

    ysq = (y2 * y2).astype(bf)
    norm = jnp.dot(g_ref[...], ysq,
                   preferred_element_type=jnp.float32) + beta_ref[...]
    o_ref[...] = y2 * jax.lax.rsqrt(norm) + ident


def _pick_band(ho):
    for r in (16, 8, 4, 2, 1):
        if ho % r == 0:
            return r
    return 1


def kernel(x, mask1, mask2, w1, b1, w2, b2, ws, bs, gamma, beta):
    del mask1, mask2
    n, cin, h, w = x.shape
    cout = w1.shape[1]
    ho = (h - 1) // 2 + 1
    wo = (w - 1) // 2 + 1
    rows = _pick_band(ho)
    nb = ho // rows

    xf = x.reshape(n * cin, h * w)

    jj = jnp.arange(w)[:, None]
    ii = jnp.arange(wo)[None, :]
    smat = jnp.concatenate(
        [(jj == 2 * ii), (jj == 2 * ii + 1)], axis=1).astype(jnp.bfloat16)

    w1l = jnp.transpose(w1, (1, 0, 2)).reshape(cout, 9 * cin).astype(jnp.bfloat16)
    w2l = jnp.transpose(w2, (1, 0, 2)).reshape(cout, 9 * cout).astype(jnp.bfloat16)
    wsl = ws.astype(jnp.bfloat16)
    gl = gamma.astype(jnp.bfloat16)

    cparams = pltpu.CompilerParams(
        dimension_semantics=("parallel", "parallel"),
        vmem_limit_bytes=64 * 1024 * 1024,
    )

    def const_spec(shape):
        return pl.BlockSpec(shape, lambda i, b: (0,) * len(shape))

    r2w = 2 * rows * w

    def row_spec(off, clamp_hi):
        if clamp_hi:
            return pl.BlockSpec(
                (cin, w),
                lambda i, b, off=off: (i, jnp.minimum(2 * rows * b + off, h - 1)))
        return pl.BlockSpec(
            (cin, w),
            lambda i, b, off=off: (i, jnp.maximum(2 * rows * b + off, 0)))

    outf = pl.pallas_call(
        functools.partial(_fused_band_kernel, rows=rows, nb=nb, w=w, wo=wo,
                          neg_slope=0.01),
        out_shape=jax.ShapeDtypeStruct((n * cout, ho * wo), jnp.float32),
        grid=(n, nb),
        in_specs=[
            pl.BlockSpec((cin, r2w), lambda i, b: (i, b)),
            row_spec(-3, False), row_spec(-2, False), row_spec(-1, False),
            row_spec(2 * rows, True), row_spec(2 * rows + 1, True),
            const_spec((w, 2 * wo)),
            const_spec((cout, 9 * cin)),
            const_spec((cout, 9 * cout)),
            const_spec((cout, cin)),
            const_spec((cout, cout)),
            const_spec((cout, 1)), const_spec((cout, 1)),
            const_spec((cout, 1)), const_spec((cout, 1)),
        ],
        out_specs=pl.BlockSpec((cout, rows * wo), lambda i, b: (i, b)),
        compiler_params=cparams,
    )(xf, xf, xf, xf, xf, xf, smat, w1l, w2l, wsl, gl, b1, b2, bs, beta)

    return outf.reshape(n, cout, ho, wo)


# speedup vs baseline: 33.2073x; 1.7038x over previous
import functools

import jax
import jax.numpy as jnp
from jax.experimental import pallas as pl
from jax.experimental.pallas import tpu as pltpu


def _fused_band_kernel(x_ref, xt_ref, xb_ref,
                       s_ref, w1_ref, w2_ref, ws_ref, g_ref,
                       b1_ref, b2_ref, bs_ref, beta_ref,
                       o_ref, *, rows, nb, w, wo, neg_slope):
    b = pl.program_id(1)
    bf = jnp.bfloat16
    cin = x_ref.shape[0]

    xm = jnp.swapaxes(x_ref[...].astype(bf), 0, 1)
    xt = jnp.swapaxes(xt_ref[...].astype(bf), 0, 1)
    xb = jnp.swapaxes(xb_ref[...].astype(bf), 0, 1)
    pieces = [xt[5], xt[6], xt[7]]
    pieces += [xm[l] for l in range(2 * rows)]
    pieces += [xb[0], xb[1]]
    xs = jnp.concatenate(pieces, axis=0)
    sel = jnp.dot(xs, s_ref[...],
                  preferred_element_type=jnp.float32).astype(bf)

    zero_mask = (b > 0).astype(bf)

    def row_eo(l):
        base = (l + 3) * cin
        blk = sel[base:base + cin]
        return blk[:, 0:wo], blk[:, wo:2 * wo]

    zc = None

    def shift_r(v):
        return jnp.concatenate([zc, v[:, 0:wo - 1]], axis=1)

    zc = jnp.zeros((cin, 1), bf)
    cols = []
    for jj in range(-1, rows + 1):
        parts = []
        for kh in range(3):
            l = 2 * jj + kh - 1
            e, o = row_eo(l)
            if l == -1:
                e = e * zero_mask
                o = o * zero_mask
            parts.append(shift_r(o))
            parts.append(e)
            parts.append(o)
        cols.append(jnp.concatenate(parts, axis=0))
    rhs1 = jnp.concatenate(cols, axis=1)
    y1 = jnp.dot(w1_ref[...], rhs1,
                 preferred_element_type=jnp.float32) + b1_ref[...]
    y1 = jnp.where(y1 >= 0.0, y1, y1 * neg_slope)

    cout = y1.shape[0]
    tm = (b > 0).astype(jnp.float32)
    bm = (b < nb - 1).astype(jnp.float32)
    y1 = y1 * jnp.concatenate(
        [jnp.full((1, wo), tm), jnp.ones((1, rows * wo)),
         jnp.full((1, wo), bm)], axis=1)
    y1b = y1.astype(bf)

    zc2 = jnp.zeros((cout, 1), bf)
    cols2 = []
    for r in range(rows):
        parts = []
        for kh in range(3):
            g = y1b[:, (r + kh) * wo:(r + kh + 1) * wo]
            parts.append(jnp.concatenate([zc2, g[:, 0:wo - 1]], axis=1))
            parts.append(g)
            parts.append(jnp.concatenate([g[:, 1:wo], zc2], axis=1))
        cols2.append(jnp.concatenate(parts, axis=0))
    rhs2 = jnp.concatenate(cols2, axis=1)
    y2 = jnp.dot(w2_ref[...], rhs2,
                 preferred_element_type=jnp.float32) + b2_ref[...]

    srhs = jnp.concatenate(
        [row_eo(2 * r)[0] for r in range(rows)], axis=1)
    ident = jnp.dot(ws_ref[...], srhs,
                    preferred_element_type=jnp.float32) + bs_ref[...]

    ysq = (y2 * y2).astype(bf)
    norm = jnp.dot(g_ref[...], ysq,
                   preferred_element_type=jnp.float32) + beta_ref[...]
    res = y2 * jax.lax.rsqrt(norm) + ident
    o_ref[...] = res.reshape(cout, rows, wo)


def _pick_band(ho):
    for r in (16, 8, 4):
        if ho % r == 0:
            return r
    raise NotImplementedError("output height must be divisible by 4")


def kernel(x, mask1, mask2, w1, b1, w2, b2, ws, bs, gamma, beta):
    del mask1, mask2
    n, cin, h, w = x.shape
    cout = w1.shape[1]
    ho = (h - 1) // 2 + 1
    wo = (w - 1) // 2 + 1
    rows = _pick_band(ho)
    nb = ho // rows

    xf = x.reshape(n * cin, h, w)

    jj = jnp.arange(w)[:, None]
    ii = jnp.arange(wo)[None, :]
    smat = jnp.concatenate(
        [(jj == 2 * ii), (jj == 2 * ii + 1)], axis=1).astype(jnp.bfloat16)

    w1l = jnp.transpose(w1, (1, 0, 2)).reshape(cout, 9 * cin).astype(jnp.bfloat16)
    w2l = jnp.transpose(w2, (1, 0, 2)).reshape(cout, 9 * cout).astype(jnp.bfloat16)
    wsl = ws.astype(jnp.bfloat16)
    gl = gamma.astype(jnp.bfloat16)

    cparams = pltpu.CompilerParams(
        dimension_semantics=("parallel", "parallel"),
        vmem_limit_bytes=64 * 1024 * 1024,
    )

    def const_spec(shape):
        return pl.BlockSpec(shape, lambda i, b: (0,) * len(shape))

    u = 2 * rows // 8

    outf = pl.pallas_call(
        functools.partial(_fused_band_kernel, rows=rows, nb=nb, w=w, wo=wo,
                          neg_slope=0.01),
        out_shape=jax.ShapeDtypeStruct((n * cout, ho, wo), jnp.float32),
        grid=(n, nb),
        in_specs=[
            pl.BlockSpec((cin, 2 * rows, w), lambda i, b: (i, b, 0)),
            pl.BlockSpec((cin, 8, w),
                         lambda i, b: (i, jnp.maximum(u * b - 1, 0), 0)),
            pl.BlockSpec((cin, 8, w),
                         lambda i, b: (i, jnp.minimum(u * (b + 1), h // 8 - 1), 0)),
            const_spec((w, 2 * wo)),
            const_spec((cout, 9 * cin)),
            const_spec((cout, 9 * cout)),
            const_spec((cout, cin)),
            const_spec((cout, cout)),
            const_spec((cout, 1)), const_spec((cout, 1)),
            const_spec((cout, 1)), const_spec((cout, 1)),
        ],
        out_specs=pl.BlockSpec((cout, rows, wo), lambda i, b: (i, b, 0)),
        compiler_params=cparams,
    )(xf, xf, xf, smat, w1l, w2l, wsl, gl, b1, b2, bs, beta)

    return outf.reshape(n, cout, ho, wo)
